# baseline (speedup 1.0000x reference)
import jax
import jax.numpy as jnp
from jax.experimental import pallas as pl

_K = 1024
_D = 64
_BN = 1024


def _vq_body(hs_ref, embed_ref, out_ref):
    hs = hs_ref[0]
    emb = embed_ref[...]
    m2 = jax.lax.dot_general(
        hs * 2.0, emb, (((1,), (1,)), ((), ())),
        preferred_element_type=jnp.float32,
    )
    s = jnp.sum(hs * hs, axis=1, keepdims=True)
    c = jnp.sum(emb * emb, axis=1)[None, :]
    t = (s - m2) + c
    mn = jnp.min(t, axis=-1, keepdims=True)
    iota = jax.lax.broadcasted_iota(jnp.int32, t.shape, 1).astype(jnp.float32)
    cand = jnp.where(t == mn, iota, float(_K))
    idx = jnp.min(cand, axis=-1, keepdims=True)
    row = jnp.transpose(idx).astype(jnp.int32)
    i = pl.program_id(0)
    out_ref[pl.ds(i, 1), :] = row


@jax.jit
def kernel(hidden_states, embed):
    shape = hidden_states.shape
    hs = hidden_states.reshape((1, -1, shape[-1]))
    n = hs.shape[1]
    grid = (n // _BN,)
    idx = pl.pallas_call(
        _vq_body,
        grid=grid,
        in_specs=[
            pl.BlockSpec((1, _BN, _D), lambda i: (0, i, 0)),
            pl.BlockSpec((_K, _D), lambda i: (0, 0)),
        ],
        out_specs=pl.BlockSpec((n // _BN, _BN), lambda i: (0, 0)),
        out_shape=jax.ShapeDtypeStruct((n // _BN, _BN), jnp.int32),
    )(hs, embed)
    return idx.reshape(shape[:-1])


# speedup vs baseline: 1.0613x; 1.0613x over previous
import jax
import jax.numpy as jnp
from jax.experimental import pallas as pl

_K = 1024
_D = 64
_BN = 2048


def _vq_body(hs_ref, embed_ref, out_ref):
    hs = hs_ref[0]
    emb = embed_ref[...]
    m2 = jax.lax.dot_general(
        hs * 2.0, emb, (((1,), (1,)), ((), ())),
        preferred_element_type=jnp.float32,
    )
    s = jnp.sum(hs * hs, axis=1, keepdims=True)
    c = jnp.sum(emb * emb, axis=1)[None, :]
    t = (s - m2) + c
    mn = jnp.min(t, axis=-1, keepdims=True)
    iota = jax.lax.broadcasted_iota(jnp.int32, t.shape, 1).astype(jnp.float32)
    cand = jnp.where(t == mn, iota, float(_K))
    idx = jnp.min(cand, axis=-1, keepdims=True)
    row = jnp.transpose(idx).astype(jnp.int32)
    rows = row.reshape(_BN // 1024, 1024)
    i = pl.program_id(0)
    for r in range(_BN // 1024):
        out_ref[pl.ds(i * (_BN // 1024) + r, 1), :] = rows[r:r + 1]


@jax.jit
def kernel(hidden_states, embed):
    shape = hidden_states.shape
    hs = hidden_states.reshape((1, -1, shape[-1]))
    n = hs.shape[1]
    grid = (n // _BN,)
    idx = pl.pallas_call(
        _vq_body,
        grid=grid,
        in_specs=[
            pl.BlockSpec((1, _BN, _D), lambda i: (0, i, 0)),
            pl.BlockSpec((_K, _D), lambda i: (0, 0)),
        ],
        out_specs=pl.BlockSpec((n // 1024, 1024), lambda i: (0, 0)),
        out_shape=jax.ShapeDtypeStruct((n // 1024, 1024), jnp.int32),
    )(hs, embed)
    return idx.reshape(shape[:-1])


# speedup vs baseline: 1.0892x; 1.0263x over previous
import jax
import jax.numpy as jnp
from jax.experimental import pallas as pl

_K = 1024
_D = 64
_BN = 4096


def _vq_body(hs_ref, embed_ref, out_ref):
    hs = hs_ref[0]
    emb = embed_ref[...]
    m2 = jax.lax.dot_general(
        hs * 2.0, emb, (((1,), (1,)), ((), ())),
        preferred_element_type=jnp.float32,
    )
    s = jnp.sum(hs * hs, axis=1, keepdims=True)
    c = jnp.sum(emb * emb, axis=1)[None, :]
    t = (s - m2) + c
    mn = jnp.min(t, axis=-1, keepdims=True)
    iota = jax.lax.broadcasted_iota(jnp.int32, t.shape, 1).astype(jnp.float32)
    cand = jnp.where(t == mn, iota, float(_K))
    idx = jnp.min(cand, axis=-1, keepdims=True)
    row = jnp.transpose(idx).astype(jnp.int32)
    rows = row.reshape(_BN // 1024, 1024)
    i = pl.program_id(0)
    for r in range(_BN // 1024):
        out_ref[pl.ds(i * (_BN // 1024) + r, 1), :] = rows[r:r + 1]


@jax.jit
def kernel(hidden_states, embed):
    shape = hidden_states.shape
    hs = hidden_states.reshape((1, -1, shape[-1]))
    n = hs.shape[1]
    grid = (n // _BN,)
    idx = pl.pallas_call(
        _vq_body,
        grid=grid,
        in_specs=[
            pl.BlockSpec((1, _BN, _D), lambda i: (0, i, 0)),
            pl.BlockSpec((_K, _D), lambda i: (0, 0)),
        ],
        out_specs=pl.BlockSpec((n // 1024, 1024), lambda i: (0, 0)),
        out_shape=jax.ShapeDtypeStruct((n // 1024, 1024), jnp.int32),
    )(hs, embed)
    return idx.reshape(shape[:-1])


# speedup vs baseline: 1.0939x; 1.0043x over previous
import jax
import jax.numpy as jnp
from jax.experimental import pallas as pl

_K = 1024
_D = 64
_BN = 8192


def _vq_body(hs_ref, embed_ref, out_ref):
    hs = hs_ref[0]
    emb = embed_ref[...]
    m2 = jax.lax.dot_general(
        hs * 2.0, emb, (((1,), (1,)), ((), ())),
        preferred_element_type=jnp.float32,
    )
    s = jnp.sum(hs * hs, axis=1, keepdims=True)
    c = jnp.sum(emb * emb, axis=1)[None, :]
    t = (s - m2) + c
    mn = jnp.min(t, axis=-1, keepdims=True)
    iota = jax.lax.broadcasted_iota(jnp.int32, t.shape, 1).astype(jnp.float32)
    cand = jnp.where(t == mn, iota, float(_K))
    idx = jnp.min(cand, axis=-1, keepdims=True)
    row = jnp.transpose(idx).astype(jnp.int32)
    rows = row.reshape(_BN // 1024, 1024)
    i = pl.program_id(0)
    for r in range(_BN // 1024):
        out_ref[pl.ds(i * (_BN // 1024) + r, 1), :] = rows[r:r + 1]


@jax.jit
def kernel(hidden_states, embed):
    shape = hidden_states.shape
    hs = hidden_states.reshape((1, -1, shape[-1]))
    n = hs.shape[1]
    grid = (n // _BN,)
    idx = pl.pallas_call(
        _vq_body,
        grid=grid,
        in_specs=[
            pl.BlockSpec((1, _BN, _D), lambda i: (0, i, 0)),
            pl.BlockSpec((_K, _D), lambda i: (0, 0)),
        ],
        out_specs=pl.BlockSpec((n // 1024, 1024), lambda i: (0, 0)),
        out_shape=jax.ShapeDtypeStruct((n // 1024, 1024), jnp.int32),
    )(hs, embed)
    return idx.reshape(shape[:-1])


# speedup vs baseline: 1.1021x; 1.0075x over previous
import jax
import jax.numpy as jnp
from jax.experimental import pallas as pl
from jax.experimental.pallas import tpu as pltpu

_K = 1024
_D = 64
_BN = 8192


def _vq_body(hs_ref, embed_ref, out_ref):
    hs = hs_ref[0]
    emb = embed_ref[...]
    m2 = jax.lax.dot_general(
        hs * 2.0, emb, (((1,), (1,)), ((), ())),
        preferred_element_type=jnp.float32,
    )
    s = jnp.sum(hs * hs, axis=1, keepdims=True)
    c = jnp.sum(emb * emb, axis=1)[None, :]
    t = (s - m2) + c
    mn = jnp.min(t, axis=-1, keepdims=True)
    iota = jax.lax.broadcasted_iota(jnp.int32, t.shape, 1).astype(jnp.float32)
    cand = jnp.where(t == mn, iota, float(_K))
    idx = jnp.min(cand, axis=-1, keepdims=True)
    row = jnp.transpose(idx).astype(jnp.int32)
    out_ref[...] = row.reshape(_BN // 1024, 1024)


@jax.jit
def kernel(hidden_states, embed):
    shape = hidden_states.shape
    hs = hidden_states.reshape((1, -1, shape[-1]))
    n = hs.shape[1]
    grid = (n // _BN,)
    idx = pl.pallas_call(
        _vq_body,
        grid=grid,
        in_specs=[
            pl.BlockSpec((1, _BN, _D), lambda i: (0, i, 0)),
            pl.BlockSpec((_K, _D), lambda i: (0, 0)),
        ],
        out_specs=pl.BlockSpec((_BN // 1024, 1024), lambda i: (i, 0)),
        out_shape=jax.ShapeDtypeStruct((n // 1024, 1024), jnp.int32),
        compiler_params=pltpu.CompilerParams(
            dimension_semantics=("parallel",)),
    )(hs, embed)
    return idx.reshape(shape[:-1])
